# baseline (device time: 39939 ns/iter reference)
import jax
import jax.numpy as jnp
from jax import lax
from jax.experimental import pallas as pl
from jax.experimental.pallas import tpu as pltpu

N_DEV = 16
N_ROUNDS = 4
B = 8
N_OUT_CHUNKS = 4


def kernel(x):
    m, n = x.shape
    nb = m // B
    nh = nb // 2
    nc = nb // N_OUT_CHUNKS

    x3 = x.reshape(nb, B, n)

    def body(xh_ref, oh_ref, xv_ref, a_ref, ob_ref, c_ref, c2_ref, p_ref,
             acc_ref, s_ref, r_ref, send_sems, recv_sems, in_sems, out_sems):
        my_i = lax.axis_index("i")

        def rdma(r, target):
            return pltpu.make_async_remote_copy(
                src_ref=s_ref.at[r],
                dst_ref=r_ref.at[r],
                send_sem=send_sems.at[r],
                recv_sem=recv_sems.at[r],
                device_id=(target,),
                device_id_type=pl.DeviceIdType.MESH,
            )

        loads = []
        for h in range(2):
            ld = pltpu.make_async_copy(
                xh_ref.at[pl.ds(h * nh, nh)],
                xv_ref.at[pl.ds(h * nh, nh)],
                in_sems.at[h],
            )
            ld.start()
            loads.append(ld)

        def prodn(ref, offs, size, mid=None):
            acc = None
            for o in offs:
                t = (ref[pl.ds(o, size), :, :] if mid is None
                     else ref[pl.ds(o, size), mid, :])
                acc = t if acc is None else acc * t
            return acc

        for h in range(2):
            loads[h].wait()
            base = h * nh
            a_ref[pl.ds(h * 32, 32), :, :] = prodn(
                xv_ref, [base + 32 * k for k in range(8)], 32
            )
        a_ref[pl.ds(0, 8), :, :] = prodn(a_ref, [8 * k for k in range(8)], 8)
        a_ref[pl.ds(0, 1), :, :] = prodn(a_ref, list(range(8)), 1)
        pv = None
        for j in range(B):
            t = a_ref[pl.ds(0, 1), pl.ds(j, 1), :]
            pv = t if pv is None else pv * t
        p_ref[...] = pv.reshape(1, n)
        acc_ref[...] = p_ref[...]

        def inblock_step(srcb, dstb, s):
            dstb[:, pl.ds(0, s), :] = srcb[:, pl.ds(0, s), :]
            dstb[:, pl.ds(s, B - s), :] = (
                srcb[:, pl.ds(s, B - s), :] * srcb[:, pl.ds(0, B - s), :]
            )

        def send_round(r, d):
            @pl.when(my_i + d < N_DEV)
            def _():
                s_ref[r, :, :] = acc_ref[...]
                rdma(r, my_i + d).start()

        def recv_round(r, d):
            @pl.when(my_i >= d)
            def _():
                rdma(r, my_i - d).wait_recv()
                acc_ref[...] = acc_ref[...] * r_ref[r, :, :]

        send_round(0, 1)
        inblock_step(xv_ref, a_ref, 1)
        recv_round(0, 1)

        send_round(1, 2)
        inblock_step(a_ref, xv_ref, 2)
        recv_round(1, 2)

        send_round(2, 4)
        inblock_step(xv_ref, a_ref, 4)
        recv_round(2, 4)

        send_round(3, 8)
        c_ref[...] = a_ref[:, pl.ds(B - 1, 1), :].reshape(nb, n)
        srcb, dstb = c_ref, c2_ref
        s = 1
        while s < nb:
            dstb[pl.ds(0, s), :] = srcb[pl.ds(0, s), :]
            dstb[pl.ds(s, nb - s), :] = (
                srcb[pl.ds(s, nb - s), :] * srcb[pl.ds(0, nb - s), :]
            )
            srcb, dstb = dstb, srcb
            s *= 2
        sc = srcb
        wb = dstb
        recv_round(3, 8)

        e = acc_ref[...] / p_ref[...]
        wb[pl.ds(0, 1), :] = e
        wb[pl.ds(1, nb - 1), :] = sc[pl.ds(0, nb - 1), :] * e

        stores = []
        for k in range(N_OUT_CHUNKS):
            c0 = k * nc
            w3 = wb[pl.ds(c0, nc), :].reshape(nc, 1, n)
            ob_ref[pl.ds(c0, nc), :, :] = (
                a_ref[pl.ds(c0, nc), :, :] * w3
            ).astype(jnp.bfloat16)
            st = pltpu.make_async_copy(
                ob_ref.at[pl.ds(c0, nc)],
                oh_ref.at[pl.ds(c0, nc)],
                out_sems.at[k],
            )
            st.start()
            stores.append(st)

        for r in range(N_ROUNDS):
            d = 1 << r

            @pl.when(my_i + d < N_DEV)
            def _(r=r, d=d):
                rdma(r, my_i + d).wait_send()

        for st in stores:
            st.wait()

    out3 = pl.pallas_call(
        body,
        out_shape=jax.ShapeDtypeStruct((nb, B, n), jnp.bfloat16),
        in_specs=[pl.BlockSpec(memory_space=pltpu.MemorySpace.HBM)],
        out_specs=pl.BlockSpec(memory_space=pltpu.MemorySpace.HBM),
        scratch_shapes=[
            pltpu.VMEM((nb, B, n), jnp.float32),
            pltpu.VMEM((nb, B, n), jnp.float32),
            pltpu.VMEM((nb, B, n), jnp.bfloat16),
            pltpu.VMEM((nb, n), jnp.float32),
            pltpu.VMEM((nb, n), jnp.float32),
            pltpu.VMEM((1, n), jnp.float32),
            pltpu.VMEM((1, n), jnp.float32),
            pltpu.VMEM((N_ROUNDS, 1, n), jnp.float32),
            pltpu.VMEM((N_ROUNDS, 1, n), jnp.float32),
            pltpu.SemaphoreType.DMA((N_ROUNDS,)),
            pltpu.SemaphoreType.DMA((N_ROUNDS,)),
            pltpu.SemaphoreType.DMA((2,)),
            pltpu.SemaphoreType.DMA((N_OUT_CHUNKS,)),
        ],
        compiler_params=pltpu.CompilerParams(
            vmem_limit_bytes=100 * 1024 * 1024,
        ),
    )(x3)
    return out3.reshape(m, n)


# device time: 38327 ns/iter; 1.0421x vs baseline; 1.0421x over previous
import jax
import jax.numpy as jnp
from jax import lax
from jax.experimental import pallas as pl
from jax.experimental.pallas import tpu as pltpu

N_DEV = 16
N_ROUNDS = 4
B = 8
N_OUT_CHUNKS = 4


def kernel(x):
    m, n = x.shape
    nb = m // B
    nh = nb // 2
    nc = nb // N_OUT_CHUNKS

    x3 = x.reshape(nb, B, n)

    def body(xh_ref, oh_ref, xv_ref, a_ref, ob_ref, c_ref, c2_ref, p_ref,
             acc_ref, s_ref, r_ref, send_sems, recv_sems, in_sems, out_sems):
        my_i = lax.axis_index("i")

        def rdma(r, target):
            return pltpu.make_async_remote_copy(
                src_ref=s_ref.at[r],
                dst_ref=r_ref.at[r],
                send_sem=send_sems.at[r],
                recv_sem=recv_sems.at[r],
                device_id=(target,),
                device_id_type=pl.DeviceIdType.MESH,
            )

        loads = []
        for h in range(2):
            ld = pltpu.make_async_copy(
                xh_ref.at[pl.ds(h * nh, nh)],
                xv_ref.at[pl.ds(h * nh, nh)],
                in_sems.at[h],
            )
            ld.start()
            loads.append(ld)

        def prodn(ref, offs, size, mid=None):
            acc = None
            for o in offs:
                t = (ref[pl.ds(o, size), :, :] if mid is None
                     else ref[pl.ds(o, size), mid, :])
                acc = t if acc is None else acc * t
            return acc

        for h in range(2):
            loads[h].wait()
            base = h * nh
            a_ref[pl.ds(h * 32, 32), :, :] = prodn(
                xv_ref, [base + 32 * k for k in range(8)], 32
            )
        a_ref[pl.ds(0, 8), :, :] = prodn(a_ref, [8 * k for k in range(8)], 8)
        a_ref[pl.ds(0, 1), :, :] = prodn(a_ref, list(range(8)), 1)
        pv = None
        for j in range(B):
            t = a_ref[pl.ds(0, 1), pl.ds(j, 1), :]
            pv = t if pv is None else pv * t
        p_ref[...] = pv.reshape(1, n)
        acc_ref[...] = p_ref[...]

        def inblock_half(h):
            base = h * nh
            parts = []
            cur = None
            for j in range(B):
                t = xv_ref[pl.ds(base, nh), pl.ds(j, 1), :]
                cur = t if cur is None else cur * t
                parts.append(cur)
            a_ref[pl.ds(base, nh), :, :] = jnp.concatenate(parts, axis=1)
            c_ref[pl.ds(base, nh), :] = cur.reshape(nh, n)

        def send_round(r, d):
            @pl.when(my_i + d < N_DEV)
            def _():
                s_ref[r, :, :] = acc_ref[...]
                rdma(r, my_i + d).start()

        def recv_round(r, d):
            @pl.when(my_i >= d)
            def _():
                rdma(r, my_i - d).wait_recv()
                acc_ref[...] = acc_ref[...] * r_ref[r, :, :]

        def carry_step(srcb, dstb, s):
            dstb[pl.ds(0, s), :] = srcb[pl.ds(0, s), :]
            dstb[pl.ds(s, nb - s), :] = (
                srcb[pl.ds(s, nb - s), :] * srcb[pl.ds(0, nb - s), :]
            )

        send_round(0, 1)
        inblock_half(0)
        recv_round(0, 1)

        send_round(1, 2)
        inblock_half(1)
        recv_round(1, 2)

        send_round(2, 4)
        srcb, dstb = c_ref, c2_ref
        for s in [1, 2, 4, 8, 16]:
            carry_step(srcb, dstb, s)
            srcb, dstb = dstb, srcb
        recv_round(2, 4)

        send_round(3, 8)
        for s in [32, 64, 128, 256]:
            carry_step(srcb, dstb, s)
            srcb, dstb = dstb, srcb
        sc = srcb
        wb = dstb
        recv_round(3, 8)

        e = acc_ref[...] / p_ref[...]
        wb[pl.ds(0, 1), :] = e
        wb[pl.ds(1, nb - 1), :] = sc[pl.ds(0, nb - 1), :] * e

        stores = []
        for k in range(N_OUT_CHUNKS):
            c0 = k * nc
            w3 = wb[pl.ds(c0, nc), :].reshape(nc, 1, n)
            ob_ref[pl.ds(c0, nc), :, :] = (
                a_ref[pl.ds(c0, nc), :, :] * w3
            ).astype(jnp.bfloat16)
            st = pltpu.make_async_copy(
                ob_ref.at[pl.ds(c0, nc)],
                oh_ref.at[pl.ds(c0, nc)],
                out_sems.at[k],
            )
            st.start()
            stores.append(st)

        for r in range(N_ROUNDS):
            d = 1 << r

            @pl.when(my_i + d < N_DEV)
            def _(r=r, d=d):
                rdma(r, my_i + d).wait_send()

        for st in stores:
            st.wait()

    out3 = pl.pallas_call(
        body,
        out_shape=jax.ShapeDtypeStruct((nb, B, n), jnp.bfloat16),
        in_specs=[pl.BlockSpec(memory_space=pltpu.MemorySpace.HBM)],
        out_specs=pl.BlockSpec(memory_space=pltpu.MemorySpace.HBM),
        scratch_shapes=[
            pltpu.VMEM((nb, B, n), jnp.float32),
            pltpu.VMEM((nb, B, n), jnp.float32),
            pltpu.VMEM((nb, B, n), jnp.bfloat16),
            pltpu.VMEM((nb, n), jnp.float32),
            pltpu.VMEM((nb, n), jnp.float32),
            pltpu.VMEM((1, n), jnp.float32),
            pltpu.VMEM((1, n), jnp.float32),
            pltpu.VMEM((N_ROUNDS, 1, n), jnp.float32),
            pltpu.VMEM((N_ROUNDS, 1, n), jnp.float32),
            pltpu.SemaphoreType.DMA((N_ROUNDS,)),
            pltpu.SemaphoreType.DMA((N_ROUNDS,)),
            pltpu.SemaphoreType.DMA((2,)),
            pltpu.SemaphoreType.DMA((N_OUT_CHUNKS,)),
        ],
        compiler_params=pltpu.CompilerParams(
            vmem_limit_bytes=100 * 1024 * 1024,
        ),
    )(x3)
    return out3.reshape(m, n)


# device time: 33957 ns/iter; 1.1762x vs baseline; 1.1287x over previous
import jax
import jax.numpy as jnp
from jax import lax
from jax.experimental import pallas as pl
from jax.experimental.pallas import tpu as pltpu

N_DEV = 16
N_ROUNDS = 4
B = 8
N_OUT_CHUNKS = 4


def kernel(x):
    m, n = x.shape
    nb = m // B
    nh = nb // 2
    nc = nb // N_OUT_CHUNKS

    x3 = x.reshape(nb, B, n)

    def body(xh_ref, oh_ref, xv_ref, a_ref, ob_ref, c_ref, c2_ref, p_ref,
             acc_ref, s_ref, r_ref, send_sems, recv_sems, in_sems, out_sems):
        my_i = lax.axis_index("i")

        barrier_sem = pltpu.get_barrier_semaphore()
        for d in (1, 2, 4, 8):
            @pl.when(my_i >= d)
            def _(d=d):
                pl.semaphore_signal(
                    barrier_sem, inc=1,
                    device_id=(my_i - d,),
                    device_id_type=pl.DeviceIdType.MESH,
                )

            @pl.when(my_i + d >= N_DEV)
            def _(d=d):
                pl.semaphore_signal(barrier_sem, inc=1)

        pl.semaphore_wait(barrier_sem, 4)

        def rdma(r, target):
            return pltpu.make_async_remote_copy(
                src_ref=s_ref.at[r],
                dst_ref=r_ref.at[r],
                send_sem=send_sems.at[r],
                recv_sem=recv_sems.at[r],
                device_id=(target,),
                device_id_type=pl.DeviceIdType.MESH,
            )

        loads = []
        for h in range(2):
            ld = pltpu.make_async_copy(
                xh_ref.at[pl.ds(h * nh, nh)],
                xv_ref.at[pl.ds(h * nh, nh)],
                in_sems.at[h],
            )
            ld.start()
            loads.append(ld)

        def prodn(ref, offs, size, mid=None):
            acc = None
            for o in offs:
                t = (ref[pl.ds(o, size), :, :] if mid is None
                     else ref[pl.ds(o, size), mid, :])
                acc = t if acc is None else acc * t
            return acc

        for h in range(2):
            loads[h].wait()
            base = h * nh
            a_ref[pl.ds(h * 32, 32), :, :] = prodn(
                xv_ref, [base + 32 * k for k in range(8)], 32
            )
        a_ref[pl.ds(0, 8), :, :] = prodn(a_ref, [8 * k for k in range(8)], 8)
        a_ref[pl.ds(0, 1), :, :] = prodn(a_ref, list(range(8)), 1)
        pv = None
        for j in range(B):
            t = a_ref[pl.ds(0, 1), pl.ds(j, 1), :]
            pv = t if pv is None else pv * t
        p_ref[...] = pv.reshape(1, n)
        acc_ref[...] = p_ref[...]

        def inblock_half(h):
            base = h * nh
            parts = []
            cur = None
            for j in range(B):
                t = xv_ref[pl.ds(base, nh), pl.ds(j, 1), :]
                cur = t if cur is None else cur * t
                parts.append(cur)
            a_ref[pl.ds(base, nh), :, :] = jnp.concatenate(parts, axis=1)
            c_ref[pl.ds(base, nh), :] = cur.reshape(nh, n)

        def send_round(r, d):
            @pl.when(my_i + d < N_DEV)
            def _():
                s_ref[r, :, :] = acc_ref[...]
                rdma(r, my_i + d).start()

        def recv_round(r, d):
            @pl.when(my_i >= d)
            def _():
                rdma(r, my_i - d).wait_recv()
                acc_ref[...] = acc_ref[...] * r_ref[r, :, :]

        def carry_step(srcb, dstb, s):
            dstb[pl.ds(0, s), :] = srcb[pl.ds(0, s), :]
            dstb[pl.ds(s, nb - s), :] = (
                srcb[pl.ds(s, nb - s), :] * srcb[pl.ds(0, nb - s), :]
            )

        send_round(0, 1)
        inblock_half(0)
        recv_round(0, 1)

        send_round(1, 2)
        inblock_half(1)
        recv_round(1, 2)

        send_round(2, 4)
        srcb, dstb = c_ref, c2_ref
        for s in [1, 2, 4, 8, 16]:
            carry_step(srcb, dstb, s)
            srcb, dstb = dstb, srcb
        recv_round(2, 4)

        send_round(3, 8)
        for s in [32, 64, 128, 256]:
            carry_step(srcb, dstb, s)
            srcb, dstb = dstb, srcb
        sc = srcb
        wb = dstb
        recv_round(3, 8)

        e = acc_ref[...] / p_ref[...]
        wb[pl.ds(0, 1), :] = e
        wb[pl.ds(1, nb - 1), :] = sc[pl.ds(0, nb - 1), :] * e

        stores = []
        for k in range(N_OUT_CHUNKS):
            c0 = k * nc
            w3 = wb[pl.ds(c0, nc), :].reshape(nc, 1, n)
            ob_ref[pl.ds(c0, nc), :, :] = (
                a_ref[pl.ds(c0, nc), :, :] * w3
            ).astype(jnp.bfloat16)
            st = pltpu.make_async_copy(
                ob_ref.at[pl.ds(c0, nc)],
                oh_ref.at[pl.ds(c0, nc)],
                out_sems.at[k],
            )
            st.start()
            stores.append(st)

        for r in range(N_ROUNDS):
            d = 1 << r

            @pl.when(my_i + d < N_DEV)
            def _(r=r, d=d):
                rdma(r, my_i + d).wait_send()

        for st in stores:
            st.wait()

    out3 = pl.pallas_call(
        body,
        out_shape=jax.ShapeDtypeStruct((nb, B, n), jnp.bfloat16),
        in_specs=[pl.BlockSpec(memory_space=pltpu.MemorySpace.HBM)],
        out_specs=pl.BlockSpec(memory_space=pltpu.MemorySpace.HBM),
        scratch_shapes=[
            pltpu.VMEM((nb, B, n), jnp.float32),
            pltpu.VMEM((nb, B, n), jnp.float32),
            pltpu.VMEM((nb, B, n), jnp.bfloat16),
            pltpu.VMEM((nb, n), jnp.float32),
            pltpu.VMEM((nb, n), jnp.float32),
            pltpu.VMEM((1, n), jnp.float32),
            pltpu.VMEM((1, n), jnp.float32),
            pltpu.VMEM((N_ROUNDS, 1, n), jnp.float32),
            pltpu.VMEM((N_ROUNDS, 1, n), jnp.float32),
            pltpu.SemaphoreType.DMA((N_ROUNDS,)),
            pltpu.SemaphoreType.DMA((N_ROUNDS,)),
            pltpu.SemaphoreType.DMA((2,)),
            pltpu.SemaphoreType.DMA((N_OUT_CHUNKS,)),
        ],
        compiler_params=pltpu.CompilerParams(
            vmem_limit_bytes=100 * 1024 * 1024,
            collective_id=0,
        ),
    )(x3)
    return out3.reshape(m, n)
